# baseline (speedup 1.0000x reference)
import functools

import jax
import jax.numpy as jnp
from jax import lax
from jax.experimental import pallas as pl
from jax.experimental.pallas import tpu as pltpu
from jax.experimental.pallas import tpu_sc as plsc

K = 15
KP = 16
KH = 8
NC, NS, LANES = 2, 16, 16
C = 16
RPT = 632
BN_ROWS = 400


def _ceil_to(v, m):
    return (v + m - 1) // m * m


def _mm_body(h_ref, w_ref, wr_ref, b_ref, y_ref, r_ref):
    h = h_ref[...]
    y_ref[...] = jnp.dot(h, w_ref[...], preferred_element_type=jnp.float32)
    r_ref[...] = (
        jnp.dot(h, wr_ref[...], preferred_element_type=jnp.float32) + b_ref[...]
    )


@functools.lru_cache(maxsize=None)
def _mm_call(n, ci_p, co_p):
    kw = KH * co_p
    nb = n // BN_ROWS
    return pl.pallas_call(
        _mm_body,
        grid=(NC, nb),
        in_specs=[
            pl.BlockSpec((BN_ROWS, ci_p), lambda c, i: (i, 0)),
            pl.BlockSpec((ci_p, kw), lambda c, i: (0, c)),
            pl.BlockSpec((ci_p, co_p), lambda c, i: (0, 0)),
            pl.BlockSpec((1, co_p), lambda c, i: (0, 0)),
        ],
        out_specs=[
            pl.BlockSpec((BN_ROWS, kw), lambda c, i: (c * nb + i, 0)),
            pl.BlockSpec((BN_ROWS, co_p), lambda c, i: (i, 0)),
        ],
        out_shape=[
            jax.ShapeDtypeStruct((NC * n, kw), jnp.float32),
            jax.ShapeDtypeStruct((n, co_p), jnp.float32),
        ],
    )


def _post_body(n, last, a_ref, r_ref, g_ref, b_ref, o_ref):
    t = a_ref[0, :n, :] + a_ref[1, :n, :] + r_ref[...]
    if last:
        o_ref[...] = t
        return
    t = jnp.where(t > 0, t, jnp.exp(t) - 1.0)
    m = jnp.mean(t, axis=0, keepdims=True)
    v = jnp.mean((t - m) * (t - m), axis=0, keepdims=True)
    o_ref[...] = (t - m) * lax.rsqrt(v + 1e-5) * g_ref[...] + b_ref[...]


@functools.lru_cache(maxsize=None)
def _post_call(n, n_agg, co_p, last):
    return pl.pallas_call(
        functools.partial(_post_body, n, last),
        in_specs=[
            pl.BlockSpec((2, n_agg, co_p), lambda: (0, 0, 0)),
            pl.BlockSpec((n, co_p), lambda: (0, 0)),
            pl.BlockSpec((1, co_p), lambda: (0, 0)),
            pl.BlockSpec((1, co_p), lambda: (0, 0)),
        ],
        out_specs=pl.BlockSpec((n, co_p), lambda: (0, 0)),
        out_shape=jax.ShapeDtypeStruct((n, co_p), jnp.float32),
    )


@functools.lru_cache(maxsize=None)
def _sc_call(co_p, n, n_agg):
    kw = KH * co_p
    nj = co_p // LANES
    mesh = plsc.VectorSubcoreMesh(core_axis_name="c", subcore_axis_name="s")

    @functools.partial(
        pl.kernel,
        out_type=jax.ShapeDtypeStruct((NC, n_agg, co_p), jnp.float32),
        mesh=mesh,
        scratch_types=[
            pltpu.VMEM((1, 8, LANES), jnp.float32),
            pltpu.VMEM((1, 8, LANES), jnp.float32),
            pltpu.VMEM((1, 8, LANES), jnp.float32),
            pltpu.VMEM((C,), jnp.int32),
            pltpu.VMEM((C,), jnp.int32),
            pltpu.VMEM((C,), jnp.int32),
            pltpu.VMEM((C, kw), jnp.float32),
            pltpu.VMEM((C, kw), jnp.float32),
            pltpu.VMEM((C, kw), jnp.float32),
            pltpu.VMEM((RPT + 8, co_p), jnp.float32),
            pltpu.VMEM((1, 8, LANES), jnp.float32),
            pltpu.VMEM((1, LANES), jnp.int32),
            pltpu.SemaphoreType.DMA,
            pltpu.SemaphoreType.DMA,
            pltpu.SemaphoreType.DMA,
            pltpu.SemaphoreType.DMA,
            pltpu.SemaphoreType.DMA,
            pltpu.SemaphoreType.DMA,
            pltpu.SemaphoreType.DMA,
            pltpu.SemaphoreType.DMA,
            pltpu.SemaphoreType.DMA,
        ],
    )
    def k(y_hbm, src_hbm, ed_hbm, mg_hbm, off_hbm, z_hbm, out_hbm,
          ed0, ed1, ed2, sb0, sb1, sb2, yv0, yv1, yv2, agg, mgv, offsv,
          se0, se1, se2, ss0, ss1, ss2, sg0, sg1, sg2):
        c = lax.axis_index("c")
        s = lax.axis_index("s")
        eds = (ed0, ed1, ed2)
        sbs = (sb0, sb1, sb2)
        yvs = (yv0, yv1, yv2)
        ses = (se0, se1, se2)
        sss = (ss0, ss1, ss2)
        sgs = (sg0, sg1, sg2)
        pltpu.sync_copy(z_hbm, agg)
        pltpu.sync_copy(mg_hbm.at[pl.ds(c, 1)], mgv)
        pltpu.sync_copy(off_hbm.at[pl.ds(s, 1)], offsv)
        mu0, mu1, mu2 = mgv[0, 0, :], mgv[0, 1, :], mgv[0, 2, :]
        a0, a1, a2 = mgv[0, 3, :], mgv[0, 4, :], mgv[0, 5, :]
        io = lax.iota(jnp.int32, LANES)
        ov = offsv[0, :]
        start = ov[0]
        end = ov[1]
        base = start - lax.rem(start, 16)
        nch = lax.div(end - base + (C - 1), C)

        def issue_ed(j, b):
            cb16 = lax.div(base, 16) + j
            pltpu.async_copy(ed_hbm.at[pl.ds(cb16, 1)], eds[b], ses[b])

        def issue_src(j, b):
            cb = pl.multiple_of(base + j * C, 16)
            pltpu.async_copy(src_hbm.at[c, pl.ds(cb, C)], sbs[b], sss[b])

        def issue_gather(j, b):
            pltpu.async_copy(y_hbm.at[sbs[b]], yvs[b], sgs[b])

        def compute(j, b):
            cb = base + j * C
            ed = eds[b]
            yv = yvs[b]
            dvec = ed[0, 1, :].astype(jnp.int32)
            ev0 = ed[0, 2, :]
            ev1 = ed[0, 3, :]
            ev2 = ed[0, 4, :]
            gi = cb + io
            ok = (gi >= start) & (gi < end)
            dl = jnp.where(ok, dvec - s * RPT, RPT)
            for l in range(C):
                d0 = ev0[l] - mu0
                d1 = ev1[l] - mu1
                d2 = ev2[l] - mu2
                g = jnp.exp(-(d0 * d0 * a0 + d1 * d1 * a1 + d2 * d2 * a2))
                dr = dl[l]
                for jj in range(nj):
                    acc = g[0] * yv[l, pl.ds(jj * LANES, LANES)]
                    for kk in range(1, KH):
                        acc = acc + g[kk] * yv[
                            l, pl.ds(kk * co_p + jj * LANES, LANES)
                        ]
                    plsc.addupdate(agg.at[dr, pl.ds(jj * LANES, LANES)], acc)

        def wait_ed(b):
            pltpu.make_async_copy(ed_hbm.at[pl.ds(0, 1)], eds[b], ses[b]).wait()

        def wait_src(b):
            pltpu.make_async_copy(
                src_hbm.at[0, pl.ds(0, C)], sbs[b], sss[b]
            ).wait()

        def wait_gather(b):
            pltpu.make_async_copy(y_hbm.at[pl.ds(0, C)], yvs[b], sgs[b]).wait()

        for b in range(3):
            issue_src(b, b)
            issue_ed(b, b)
        wait_src(0)
        issue_gather(0, 0)
        wait_src(1)
        issue_gather(1, 1)

        def group_body(gidx, carry):
            j0 = gidx * 3
            for b in range(3):
                j = j0 + b
                bn = (b + 2) % 3
                wait_src(bn)
                issue_gather(j + 2, bn)
                wait_gather(b)
                issue_src(j + 3, b)
                wait_ed(b)
                compute(j, b)
                issue_ed(j + 3, b)
            return carry

        ngrp = lax.max(1, lax.div(nch + 2, 3))
        lax.fori_loop(0, ngrp, group_body, 0)
        wait_src(2)
        wait_gather(0)
        wait_gather(1)
        wait_ed(0)
        wait_ed(1)
        wait_ed(2)

        pltpu.sync_copy(
            agg.at[pl.ds(0, RPT)], out_hbm.at[c, pl.ds(s * RPT, RPT)]
        )

    return k


def kernel(x, edge_index, edge_attr, params):
    n = x.shape[0]
    e = edge_attr.shape[0]
    n_agg = NS * RPT
    e_cap = _ceil_to(e, C)
    e_alloc = e_cap + 10 * C

    npd = e_alloc - e
    src = jnp.concatenate([edge_index[0], jnp.zeros((npd,), jnp.int32)])
    dst = jnp.concatenate([edge_index[1], jnp.full((npd,), n, jnp.int32)])
    ea = jnp.zeros((e_alloc, 3), jnp.float32).at[:e].set(edge_attr)

    bucket = dst[:e_cap] // RPT
    perm = jnp.argsort(bucket, stable=True)
    perm = jnp.concatenate([perm, jnp.arange(e_cap, e_alloc, dtype=perm.dtype)])
    src = src[perm]
    dst = dst[perm]
    ea = ea[perm]
    counts = jnp.bincount(bucket, length=NS)
    csum = jnp.concatenate(
        [jnp.zeros((1,), jnp.int32), jnp.cumsum(counts).astype(jnp.int32)]
    )
    off = (
        jnp.zeros((NS, LANES), jnp.int32)
        .at[:, 0].set(csum[:NS])
        .at[:, 1].set(csum[1 : NS + 1])
    )
    src2 = jnp.stack([src, src + n])
    ed = jnp.zeros((e_alloc // 16, 8, LANES), jnp.float32)
    ed = ed.at[:, 1].set(dst.astype(jnp.float32).reshape(-1, 16))
    ed = ed.at[:, 2].set(ea[:, 0].reshape(-1, 16))
    ed = ed.at[:, 3].set(ea[:, 1].reshape(-1, 16))
    ed = ed.at[:, 4].set(ea[:, 2].reshape(-1, 16))

    h = x
    for i, p in enumerate(params):
        ci = p["W"].shape[0]
        ci_p = h.shape[1]
        co = p["root"].shape[1]
        co_p = _ceil_to(co, LANES)
        last = i == len(params) - 1

        w = p["W"].reshape(ci, K, co)
        w_pad = (
            jnp.zeros((ci_p, KP, co_p), jnp.float32)
            .at[:ci, :K, :co].set(w)
            .reshape(ci_p, KP * co_p)
        )
        wr_pad = jnp.zeros((ci_p, co_p), jnp.float32).at[:ci, :co].set(p["root"])
        b_pad = jnp.zeros((1, co_p), jnp.float32).at[0, :co].set(p["bias"])
        g_pad = jnp.zeros((1, co_p), jnp.float32).at[0, :co].set(p["gamma"])
        be_pad = jnp.zeros((1, co_p), jnp.float32).at[0, :co].set(p["beta"])
        mu_pad = jnp.zeros((KP, 3), jnp.float32).at[:K].set(p["mu"])
        a_pad = (
            jnp.zeros((KP, 3), jnp.float32)
            .at[:K].set(0.5 / (p["sigma"] ** 2 + 1e-12))
        )
        mg = jnp.concatenate(
            [
                mu_pad.T.reshape(3, NC, KH).transpose(1, 0, 2),
                a_pad.T.reshape(3, NC, KH).transpose(1, 0, 2),
            ],
            axis=1,
        )
        mg = jnp.concatenate(
            [mg, jnp.zeros((NC, 2, KH), jnp.float32)], axis=1
        )
        mg = jnp.concatenate(
            [mg, jnp.zeros((NC, 8, LANES - KH), jnp.float32)], axis=2
        )
        zeros_agg = jnp.zeros((RPT + 8, co_p), jnp.float32)

        y, r = _mm_call(n, ci_p, co_p)(h, w_pad, wr_pad, b_pad)
        aggp = _sc_call(co_p, n, n_agg)(y, src2, ed, mg, off, zeros_agg)
        h = _post_call(n, n_agg, co_p, last)(aggp, r, g_pad, be_pad)

    return h[:, : params[-1]["root"].shape[1]]


# speedup vs baseline: 1.0826x; 1.0826x over previous
import functools

import jax
import jax.numpy as jnp
from jax import lax
from jax.experimental import pallas as pl
from jax.experimental.pallas import tpu as pltpu
from jax.experimental.pallas import tpu_sc as plsc

K = 15
KP = 16
KH = 8
NC, NS, LANES = 2, 16, 16
C = 16
RPT = 632
BN_ROWS = 400


def _ceil_to(v, m):
    return (v + m - 1) // m * m


def _mm_body(h_ref, w_ref, wr_ref, b_ref, y_ref, r_ref):
    h = h_ref[...]
    y_ref[...] = jnp.dot(h, w_ref[...], preferred_element_type=jnp.float32)
    r_ref[...] = (
        jnp.dot(h, wr_ref[...], preferred_element_type=jnp.float32) + b_ref[...]
    )


@functools.lru_cache(maxsize=None)
def _mm_call(n, ci_p, co_p):
    kw = KH * co_p
    nb = n // BN_ROWS
    return pl.pallas_call(
        _mm_body,
        grid=(NC, nb),
        in_specs=[
            pl.BlockSpec((BN_ROWS, ci_p), lambda c, i: (i, 0)),
            pl.BlockSpec((ci_p, kw), lambda c, i: (0, c)),
            pl.BlockSpec((ci_p, co_p), lambda c, i: (0, 0)),
            pl.BlockSpec((1, co_p), lambda c, i: (0, 0)),
        ],
        out_specs=[
            pl.BlockSpec((BN_ROWS, kw), lambda c, i: (c * nb + i, 0)),
            pl.BlockSpec((BN_ROWS, co_p), lambda c, i: (i, 0)),
        ],
        out_shape=[
            jax.ShapeDtypeStruct((NC * n, kw), jnp.float32),
            jax.ShapeDtypeStruct((n, co_p), jnp.float32),
        ],
    )


def _post_body(n, last, a_ref, r_ref, g_ref, b_ref, o_ref):
    t = a_ref[0, :n, :] + a_ref[1, :n, :] + r_ref[...]
    if last:
        o_ref[...] = t
        return
    t = jnp.where(t > 0, t, jnp.exp(t) - 1.0)
    m = jnp.mean(t, axis=0, keepdims=True)
    v = jnp.mean((t - m) * (t - m), axis=0, keepdims=True)
    o_ref[...] = (t - m) * lax.rsqrt(v + 1e-5) * g_ref[...] + b_ref[...]


@functools.lru_cache(maxsize=None)
def _post_call(n, n_agg, co_p, last):
    return pl.pallas_call(
        functools.partial(_post_body, n, last),
        in_specs=[
            pl.BlockSpec((2, n_agg, co_p), lambda: (0, 0, 0)),
            pl.BlockSpec((n, co_p), lambda: (0, 0)),
            pl.BlockSpec((1, co_p), lambda: (0, 0)),
            pl.BlockSpec((1, co_p), lambda: (0, 0)),
        ],
        out_specs=pl.BlockSpec((n, co_p), lambda: (0, 0)),
        out_shape=jax.ShapeDtypeStruct((n, co_p), jnp.float32),
    )


@functools.lru_cache(maxsize=None)
def _sc_call(co_p, n, n_agg):
    kw = KH * co_p
    nj = co_p // LANES
    mesh = plsc.VectorSubcoreMesh(core_axis_name="c", subcore_axis_name="s")

    @functools.partial(
        pl.kernel,
        out_type=jax.ShapeDtypeStruct((NC, n_agg, co_p), jnp.float32),
        mesh=mesh,
        scratch_types=[
            pltpu.VMEM((1, 8, LANES), jnp.float32),
            pltpu.VMEM((1, 8, LANES), jnp.float32),
            pltpu.VMEM((C,), jnp.int32),
            pltpu.VMEM((C,), jnp.int32),
            pltpu.VMEM((C, kw), jnp.float32),
            pltpu.VMEM((C, kw), jnp.float32),
            pltpu.VMEM((RPT + 8, co_p), jnp.float32),
            pltpu.VMEM((1, 8, LANES), jnp.float32),
            pltpu.VMEM((1, LANES), jnp.int32),
            pltpu.SemaphoreType.DMA,
            pltpu.SemaphoreType.DMA,
            pltpu.SemaphoreType.DMA,
            pltpu.SemaphoreType.DMA,
            pltpu.SemaphoreType.DMA,
            pltpu.SemaphoreType.DMA,
        ],
    )
    def k(y_hbm, src_hbm, ed_hbm, mg_hbm, off_hbm, z_hbm, out_hbm,
          ed0, ed1, sb0, sb1, yv0, yv1, agg, mgv, offsv,
          se0, se1, ss0, ss1, sg0, sg1):
        c = lax.axis_index("c")
        s = lax.axis_index("s")
        eds = (ed0, ed1)
        sbs = (sb0, sb1)
        yvs = (yv0, yv1)
        ses = (se0, se1)
        sss = (ss0, ss1)
        sgs = (sg0, sg1)
        pltpu.sync_copy(z_hbm, agg)
        pltpu.sync_copy(mg_hbm.at[pl.ds(c, 1)], mgv)
        pltpu.sync_copy(off_hbm.at[pl.ds(s, 1)], offsv)
        mu0, mu1, mu2 = mgv[0, 0, :], mgv[0, 1, :], mgv[0, 2, :]
        a0, a1, a2 = mgv[0, 3, :], mgv[0, 4, :], mgv[0, 5, :]
        io = lax.iota(jnp.int32, LANES)
        ov = offsv[0, :]
        start = ov[0]
        end = ov[1]
        base = start - lax.rem(start, 16)
        nch = lax.div(end - base + (C - 1), C)

        def issue_ed(j, b):
            cb16 = lax.div(base, 16) + j
            pltpu.async_copy(ed_hbm.at[pl.ds(cb16, 1)], eds[b], ses[b])

        def issue_src(j, b):
            cb = pl.multiple_of(base + j * C, 16)
            pltpu.async_copy(src_hbm.at[c, pl.ds(cb, C)], sbs[b], sss[b])

        def issue_gather(j, b):
            pltpu.async_copy(y_hbm.at[sbs[b]], yvs[b], sgs[b])

        def compute(j, b):
            cb = base + j * C
            ed = eds[b]
            yv = yvs[b]
            dvec = ed[0, 1, :].astype(jnp.int32)
            ev0 = ed[0, 2, :]
            ev1 = ed[0, 3, :]
            ev2 = ed[0, 4, :]
            gi = cb + io
            ok = (gi >= start) & (gi < end)
            dl = jnp.where(ok, dvec - s * RPT, RPT)
            for l in range(C):
                d0 = ev0[l] - mu0
                d1 = ev1[l] - mu1
                d2 = ev2[l] - mu2
                g = jnp.exp(-(d0 * d0 * a0 + d1 * d1 * a1 + d2 * d2 * a2))
                dr = dl[l]
                for jj in range(nj):
                    acc = g[0] * yv[l, pl.ds(jj * LANES, LANES)]
                    for kk in range(1, KH):
                        acc = acc + g[kk] * yv[
                            l, pl.ds(kk * co_p + jj * LANES, LANES)
                        ]
                    plsc.addupdate(agg.at[dr, pl.ds(jj * LANES, LANES)], acc)

        def wait_ed(b):
            pltpu.make_async_copy(ed_hbm.at[pl.ds(0, 1)], eds[b], ses[b]).wait()

        def wait_src(b):
            pltpu.make_async_copy(
                src_hbm.at[0, pl.ds(0, C)], sbs[b], sss[b]
            ).wait()

        def wait_gather(b):
            pltpu.make_async_copy(y_hbm.at[pl.ds(0, C)], yvs[b], sgs[b]).wait()

        issue_src(0, 0)
        issue_src(1, 1)
        issue_ed(0, 0)
        issue_ed(1, 1)
        wait_src(0)
        issue_gather(0, 0)

        def group_body(gidx, carry):
            j0 = gidx * 2
            wait_src(1)
            issue_gather(j0 + 1, 1)
            wait_gather(0)
            issue_src(j0 + 2, 0)
            wait_ed(0)
            compute(j0, 0)
            issue_ed(j0 + 2, 0)
            wait_src(0)
            issue_gather(j0 + 2, 0)
            wait_gather(1)
            issue_src(j0 + 3, 1)
            wait_ed(1)
            compute(j0 + 1, 1)
            issue_ed(j0 + 3, 1)
            return carry

        ngrp = lax.max(1, lax.div(nch + 1, 2))
        lax.fori_loop(0, ngrp, group_body, 0)
        wait_ed(0)
        wait_ed(1)
        wait_src(1)
        wait_gather(0)

        pltpu.sync_copy(
            agg.at[pl.ds(0, RPT)], out_hbm.at[c, pl.ds(s * RPT, RPT)]
        )

    return k


def kernel(x, edge_index, edge_attr, params):
    n = x.shape[0]
    e = edge_attr.shape[0]
    n_agg = NS * RPT
    e_cap = _ceil_to(e, C)
    e_alloc = e_cap + 10 * C

    npd = e_alloc - e
    src = jnp.concatenate([edge_index[0], jnp.zeros((npd,), jnp.int32)])
    dst = jnp.concatenate([edge_index[1], jnp.full((npd,), n, jnp.int32)])
    ea = jnp.zeros((e_alloc, 3), jnp.float32).at[:e].set(edge_attr)

    bucket = dst[:e_cap] // RPT
    perm = jnp.argsort(bucket, stable=True)
    perm = jnp.concatenate([perm, jnp.arange(e_cap, e_alloc, dtype=perm.dtype)])
    src = src[perm]
    dst = dst[perm]
    ea = ea[perm]
    counts = jnp.bincount(bucket, length=NS)
    csum = jnp.concatenate(
        [jnp.zeros((1,), jnp.int32), jnp.cumsum(counts).astype(jnp.int32)]
    )
    off = (
        jnp.zeros((NS, LANES), jnp.int32)
        .at[:, 0].set(csum[:NS])
        .at[:, 1].set(csum[1 : NS + 1])
    )
    src2 = jnp.stack([src, src + n])
    ed = jnp.zeros((e_alloc // 16, 8, LANES), jnp.float32)
    ed = ed.at[:, 1].set(dst.astype(jnp.float32).reshape(-1, 16))
    ed = ed.at[:, 2].set(ea[:, 0].reshape(-1, 16))
    ed = ed.at[:, 3].set(ea[:, 1].reshape(-1, 16))
    ed = ed.at[:, 4].set(ea[:, 2].reshape(-1, 16))

    h = x
    for i, p in enumerate(params):
        ci = p["W"].shape[0]
        ci_p = h.shape[1]
        co = p["root"].shape[1]
        co_p = _ceil_to(co, LANES)
        last = i == len(params) - 1

        w = p["W"].reshape(ci, K, co)
        w_pad = (
            jnp.zeros((ci_p, KP, co_p), jnp.float32)
            .at[:ci, :K, :co].set(w)
            .reshape(ci_p, KP * co_p)
        )
        wr_pad = jnp.zeros((ci_p, co_p), jnp.float32).at[:ci, :co].set(p["root"])
        b_pad = jnp.zeros((1, co_p), jnp.float32).at[0, :co].set(p["bias"])
        g_pad = jnp.zeros((1, co_p), jnp.float32).at[0, :co].set(p["gamma"])
        be_pad = jnp.zeros((1, co_p), jnp.float32).at[0, :co].set(p["beta"])
        mu_pad = jnp.zeros((KP, 3), jnp.float32).at[:K].set(p["mu"])
        a_pad = (
            jnp.zeros((KP, 3), jnp.float32)
            .at[:K].set(0.5 / (p["sigma"] ** 2 + 1e-12))
        )
        mg = jnp.concatenate(
            [
                mu_pad.T.reshape(3, NC, KH).transpose(1, 0, 2),
                a_pad.T.reshape(3, NC, KH).transpose(1, 0, 2),
            ],
            axis=1,
        )
        mg = jnp.concatenate(
            [mg, jnp.zeros((NC, 2, KH), jnp.float32)], axis=1
        )
        mg = jnp.concatenate(
            [mg, jnp.zeros((NC, 8, LANES - KH), jnp.float32)], axis=2
        )
        zeros_agg = jnp.zeros((RPT + 8, co_p), jnp.float32)

        y, r = _mm_call(n, ci_p, co_p)(h, w_pad, wr_pad, b_pad)
        aggp = _sc_call(co_p, n, n_agg)(y, src2, ed, mg, off, zeros_agg)
        h = _post_call(n, n_agg, co_p, last)(aggp, r, g_pad, be_pad)

    return h[:, : params[-1]["root"].shape[1]]
